# initial kernel scaffold (unmeasured)
import jax
import jax.numpy as jnp
from jax import lax
from jax.experimental import pallas as pl
from jax.experimental.pallas import tpu as pltpu


def kernel(
    x,
):
    def body(*refs):
        pass

    out_shape = jax.ShapeDtypeStruct(..., jnp.float32)
    return pl.pallas_call(body, out_shape=out_shape)(...)


# baseline (device time: 329482 ns/iter reference)
import jax
import jax.numpy as jnp
from jax import lax
from jax.experimental import pallas as pl
from jax.experimental.pallas import tpu as pltpu

NZ = 4


def kernel(x):
    x = x.astype(jnp.bfloat16)
    m, n = x.shape
    mc = m // NZ

    def body(x_ref, out_ref, rs_recv, rs_send,
             rs_send_sems, rs_recv_sems, ag_send_sems, ag_recv_sems):
        my_x = lax.axis_index("x")
        my_y = lax.axis_index("y")
        my_z = lax.axis_index("z")
        right = (my_z + 1) % NZ
        left = (my_z + NZ - 1) % NZ

        barrier_sem = pltpu.get_barrier_semaphore()
        for nbr in (left, right):
            pl.semaphore_signal(
                barrier_sem, inc=1,
                device_id=(my_x, my_y, nbr),
                device_id_type=pl.DeviceIdType.MESH,
            )
        pl.semaphore_wait(barrier_sem, 2)

        def xchunk(c):
            return x_ref[pl.ds(c * mc, mc), :]

        for h in range(NZ - 1):
            s_idx = (my_z + NZ - h) % NZ
            if h == 0:
                src = x_ref.at[pl.ds(s_idx * mc, mc), :]
            else:
                rs_send[h - 1] = rs_recv[h - 1] + xchunk(s_idx)
                src = rs_send.at[h - 1]
            rdma = pltpu.make_async_remote_copy(
                src_ref=src,
                dst_ref=rs_recv.at[h],
                send_sem=rs_send_sems.at[h],
                recv_sem=rs_recv_sems.at[h],
                device_id=(my_x, my_y, right),
                device_id_type=pl.DeviceIdType.MESH,
            )
            rdma.start()
            rdma.wait()

        q = (my_z + 1) % NZ
        out_ref[pl.ds(q * mc, mc), :] = rs_recv[NZ - 2] + xchunk(q)

        for h in range(NZ - 1):
            g = (my_z + 1 + NZ - h) % NZ
            rdma = pltpu.make_async_remote_copy(
                src_ref=out_ref.at[pl.ds(g * mc, mc), :],
                dst_ref=out_ref.at[pl.ds(g * mc, mc), :],
                send_sem=ag_send_sems.at[h],
                recv_sem=ag_recv_sems.at[h],
                device_id=(my_x, my_y, right),
                device_id_type=pl.DeviceIdType.MESH,
            )
            rdma.start()
            rdma.wait()

    return pl.pallas_call(
        body,
        out_shape=jax.ShapeDtypeStruct((m, n), jnp.bfloat16),
        in_specs=[pl.BlockSpec(memory_space=pltpu.VMEM)],
        out_specs=pl.BlockSpec(memory_space=pltpu.VMEM),
        scratch_shapes=[
            pltpu.VMEM((NZ - 1, mc, n), jnp.bfloat16),
            pltpu.VMEM((NZ - 2, mc, n), jnp.bfloat16),
            pltpu.SemaphoreType.DMA((NZ - 1,)),
            pltpu.SemaphoreType.DMA((NZ - 1,)),
            pltpu.SemaphoreType.DMA((NZ - 1,)),
            pltpu.SemaphoreType.DMA((NZ - 1,)),
        ],
        compiler_params=pltpu.CompilerParams(
            collective_id=0,
            vmem_limit_bytes=100 * 1024 * 1024,
        ),
    )(x)


# device time: 329140 ns/iter; 1.0010x vs baseline; 1.0010x over previous
import jax
import jax.numpy as jnp
from jax import lax
from jax.experimental import pallas as pl
from jax.experimental.pallas import tpu as pltpu

NZ = 4


def kernel(x):
    x = x.astype(jnp.bfloat16)
    m, n = x.shape
    mc = m // NZ
    hn = n // 2

    def body(x_ref, out_ref, rs_recv, rs_send,
             send_sems, recv_sems):
        my_x = lax.axis_index("x")
        my_y = lax.axis_index("y")
        my_z = lax.axis_index("z")
        right = (my_z + 1) % NZ
        left = (my_z + NZ - 1) % NZ

        barrier_sem = pltpu.get_barrier_semaphore()
        for nbr in (left, right):
            pl.semaphore_signal(
                barrier_sem, inc=1,
                device_id=(my_x, my_y, nbr),
                device_id_type=pl.DeviceIdType.MESH,
            )
        pl.semaphore_wait(barrier_sem, 2)

        def xchunk(c, d):
            return x_ref[pl.ds(c * mc, mc), pl.ds(d * hn, hn)]

        def rs_rdma(h, d, src):
            dst_z = right if d == 0 else left
            return pltpu.make_async_remote_copy(
                src_ref=src,
                dst_ref=rs_recv.at[d, h],
                send_sem=send_sems.at[d, h],
                recv_sem=recv_sems.at[d, h],
                device_id=(my_x, my_y, dst_z),
                device_id_type=pl.DeviceIdType.MESH,
            )

        for h in range(NZ - 1):
            rdmas = []
            for d in range(2):
                s_idx = (my_z + NZ - h) % NZ if d == 0 else (my_z + h) % NZ
                if h == 0:
                    rs_send[d, 0] = xchunk(s_idx, d)
                else:
                    rs_send[d, h] = rs_recv[d, h - 1] + xchunk(s_idx, d)
                rdmas.append(rs_rdma(h, d, rs_send.at[d, h]))
            for r in rdmas:
                r.start()
            for r in rdmas:
                r.wait()

        for d in range(2):
            q = (my_z + 1) % NZ if d == 0 else (my_z + NZ - 1) % NZ
            out_ref[pl.ds(q * mc, mc), pl.ds(d * hn, hn)] = (
                rs_recv[d, NZ - 2] + xchunk(q, d)
            )

        for h in range(NZ - 1):
            rdmas = []
            for d in range(2):
                g = (my_z + 1 + NZ - h) % NZ if d == 0 else (my_z + NZ - 1 + h) % NZ
                dst_z = right if d == 0 else left
                rdmas.append(pltpu.make_async_remote_copy(
                    src_ref=out_ref.at[pl.ds(g * mc, mc), pl.ds(d * hn, hn)],
                    dst_ref=out_ref.at[pl.ds(g * mc, mc), pl.ds(d * hn, hn)],
                    send_sem=send_sems.at[d, NZ - 1 + h],
                    recv_sem=recv_sems.at[d, NZ - 1 + h],
                    device_id=(my_x, my_y, dst_z),
                    device_id_type=pl.DeviceIdType.MESH,
                ))
            for r in rdmas:
                r.start()
            for r in rdmas:
                r.wait()

    return pl.pallas_call(
        body,
        out_shape=jax.ShapeDtypeStruct((m, n), jnp.bfloat16),
        in_specs=[pl.BlockSpec(memory_space=pltpu.VMEM)],
        out_specs=pl.BlockSpec(memory_space=pltpu.VMEM),
        scratch_shapes=[
            pltpu.VMEM((2, NZ - 1, mc, hn), jnp.bfloat16),
            pltpu.VMEM((2, NZ - 1, mc, hn), jnp.bfloat16),
            pltpu.SemaphoreType.DMA((2, 2 * (NZ - 1))),
            pltpu.SemaphoreType.DMA((2, 2 * (NZ - 1))),
        ],
        compiler_params=pltpu.CompilerParams(
            collective_id=0,
            vmem_limit_bytes=100 * 1024 * 1024,
        ),
    )(x)
